# baseline (speedup 1.0000x reference)
import functools

import jax
import jax.numpy as jnp
from jax import lax
from jax.experimental import pallas as pl
from jax.experimental.pallas import tpu as pltpu
from jax.experimental.pallas import tpu_sc as plsc

D = 64
B = 16384
V = 1000000
L = 16
NC = 2
NS = 16
NW = NC * NS
ROWS_PER_W = B // NW
NBLK = (V + 127) // 128
BLK_PER_W = NBLK // NW
BLK_EXTRA = NBLK - BLK_PER_W * NW
RING = 4
MAXWIN = (BLK_PER_W + 1 + RING - 1) // RING + 1
SROWS = B + 2 * L
EPS = 1e-6


def _sqrt16(x):
    i = plsc.bitcast(x, jnp.int32)
    y = plsc.bitcast((i >> 1) + jnp.int32(0x1FBD1DF5), jnp.float32)
    half = jnp.float32(0.5)
    y = half * (y + x / y)
    y = half * (y + x / y)
    y = half * (y + x / y)
    return y


def _gather_body(u_idx, i_idx, u_tabT, i_tabT, stg_u, stg_i,
                 all_idx, wl_k, wl_cb, ring, batch, kidx, sem, sem2):
    wid = lax.axis_index("s") * NC + lax.axis_index("c")
    extra = jnp.minimum(wid, BLK_EXTRA)
    lo = wid * BLK_PER_W + extra
    nblk = BLK_PER_W + jnp.where(wid < BLK_EXTRA, 1, 0)
    hi = lo + nblk
    lanes = lax.iota(jnp.int32, L)

    for tab_ref, stg_ref, idx_ref in (
        (u_tabT, stg_u, u_idx),
        (i_tabT, stg_i, i_idx),
    ):
        pltpu.sync_copy(idx_ref, all_idx)

        def scan(q, cur):
            v = all_idx[pl.ds(q * L, L)]
            blk = v >> 7
            m = (blk >= lo) & (blk < hi)
            n = plsc.all_reduce_population_count(m)[0]
            kv = q * L + lanes
            cb = (v & 127) | (blk << 8)
            plsc.store_compressed(wl_k.at[pl.ds(cur, L)], kv, mask=m)
            plsc.store_compressed(wl_cb.at[pl.ds(cur, L)], cb, mask=m)
            return cur + n

        nwl = lax.fori_loop(0, B // L, scan, jnp.int32(0))
        nwlv = (nwl + L - 1) // L

        def fire(wi):
            wst = jnp.minimum(lo + wi * RING, NBLK - RING)
            start = pl.multiple_of(wst * 128, 128)
            par = wi & 1
            for a in range(D // 8):
                pltpu.async_copy(
                    tab_ref.at[pl.ds(a * 8, 8), pl.ds(start, RING * 128)],
                    ring.at[par, a], sem)

        def drain(wi):
            par = wi & 1
            for a in range(D // 8):
                pltpu.make_async_copy(
                    tab_ref.at[pl.ds(0, 8), pl.ds(0, RING * 128)],
                    ring.at[par, a], sem).wait()

        @pl.when(lo < hi)
        def _():
            fire(jnp.int32(0))

        def window(wi, _):
            wb = lo + wi * RING
            wst = jnp.minimum(wb, NBLK - RING)
            we = jnp.minimum(wb + RING, hi)
            par = wi & 1

            @pl.when(wb < hi)
            def _():
                drain(wi)

                @pl.when(wb + RING < hi)
                def _():
                    fire(wi + 1)

                def scan_hits(q, _):
                    kv = wl_k[pl.ds(q * L, L)]
                    cb = wl_cb[pl.ds(q * L, L)]
                    blk = cb >> 8
                    valid = (q * L + lanes) < nwl
                    m = (blk >= wb) & (blk < we) & valid
                    n = plsc.all_reduce_population_count(m)[0]

                    @pl.when(n > 0)
                    def _():
                        sc_vec = jnp.where(
                            m, (blk - wst) * 128 + (cb & 255), 0)
                        ksel = jnp.where(m, kv, B + lanes)
                        pv = jnp.full((L,), par, jnp.int32)
                        for f in range(D):
                            av = jnp.full((L,), f // 8, jnp.int32)
                            bv = jnp.full((L,), f % 8, jnp.int32)
                            fv = jnp.full((L,), f, jnp.int32)
                            vals = plsc.load_gather(
                                ring, [pv, av, bv, sc_vec])
                            plsc.store_scatter(batch, [lanes, fv], vals)
                        kidx[...] = ksel
                        pltpu.async_copy(
                            batch, stg_ref.at[kidx], sem2).wait()
                    return 0

                lax.fori_loop(0, nwlv, scan_hits, 0)
            return 0

        lax.fori_loop(0, MAXWIN, window, 0)


def _dist_body(stg_u, stg_i, out_hbm, buf_u, buf_i, out_v, sem_u, sem_i):
    wid = lax.axis_index("s") * NC + lax.axis_index("c")
    base = wid * ROWS_PER_W
    lanes = lax.iota(jnp.int32, L)
    CH = 256
    for ch in range(ROWS_PER_W // CH):
        cu = pltpu.async_copy(
            stg_u.at[pl.ds(base + ch * CH, CH)], buf_u, sem_u)
        ci = pltpu.async_copy(
            stg_i.at[pl.ds(base + ch * CH, CH)], buf_i, sem_i)
        cu.wait()
        ci.wait()

        def group(g, _, ch=ch):
            rv = g * L + lanes
            acc = jnp.zeros((L,), jnp.float32)
            for f in range(D):
                fv = jnp.full((L,), f, jnp.int32)
                u = plsc.load_gather(buf_u, [rv, fv])
                v = plsc.load_gather(buf_i, [rv, fv])
                d = (u - v) + jnp.float32(EPS)
                acc = acc + d * d
            out_v[pl.ds(ch * CH + g * L, L)] = _sqrt16(acc)
            return 0

        lax.fori_loop(0, CH // L, group, 0)

    pltpu.sync_copy(out_v, out_hbm.at[pl.ds(base, ROWS_PER_W)])


_MESH = plsc.VectorSubcoreMesh(core_axis_name="c", subcore_axis_name="s")
_PARAMS = pltpu.CompilerParams(needs_layout_passes=False)

_gather_phase = functools.partial(
    pl.kernel,
    mesh=_MESH,
    out_type=(
        jax.ShapeDtypeStruct((SROWS, 128), jnp.float32),
        jax.ShapeDtypeStruct((SROWS, 128), jnp.float32),
    ),
    compiler_params=_PARAMS,
    scratch_types=[
        pltpu.VMEM((B,), jnp.int32),
        pltpu.VMEM((B + L,), jnp.int32),
        pltpu.VMEM((B + L,), jnp.int32),
        pltpu.VMEM((2, D // 8, 8, RING * 128), jnp.float32),
        pltpu.VMEM((L, 128), jnp.float32),
        pltpu.VMEM((L,), jnp.int32),
        pltpu.SemaphoreType.DMA,
        pltpu.SemaphoreType.DMA,
    ],
)(lambda u_idx, i_idx, u_tabT, i_tabT, stg_u, stg_i, *scratch:
  _gather_body(u_idx, i_idx, u_tabT, i_tabT, stg_u, stg_i, *scratch))

_dist_phase = functools.partial(
    pl.kernel,
    mesh=_MESH,
    out_type=jax.ShapeDtypeStruct((B,), jnp.float32),
    compiler_params=_PARAMS,
    scratch_types=[
        pltpu.VMEM((256, 128), jnp.float32),
        pltpu.VMEM((256, 128), jnp.float32),
        pltpu.VMEM((ROWS_PER_W,), jnp.float32),
        pltpu.SemaphoreType.DMA,
        pltpu.SemaphoreType.DMA,
    ],
)(lambda stg_u, stg_i, out, *scratch: _dist_body(stg_u, stg_i, out, *scratch))


def kernel(U, I, UEmb_weight, IEmb_weight):
    stg_u, stg_i = _gather_phase(U, I, UEmb_weight.T, IEmb_weight.T)
    return _dist_phase(stg_u, stg_i)


# speedup vs baseline: 1.1215x; 1.1215x over previous
import functools

import jax
import jax.numpy as jnp
from jax import lax
from jax.experimental import pallas as pl
from jax.experimental.pallas import tpu as pltpu
from jax.experimental.pallas import tpu_sc as plsc

D = 64
B = 16384
V = 1000000
L = 16
NC = 2
NS = 16
NW = NC * NS
ROWS_PER_W = B // NW
NBLK = (V + 127) // 128
SC_HALF = (NBLK + 1) // 2
RING = 3
_MAXBLK = (SC_HALF + NS - 1) // NS
MAXWIN = (_MAXBLK + RING - 1) // RING + 1
SROWS = B + 2 * L
EPS = 1e-6


def _sqrt16(x):
    i = plsc.bitcast(x, jnp.int32)
    y = plsc.bitcast((i >> 1) + jnp.int32(0x1FBD1DF5), jnp.float32)
    half = jnp.float32(0.5)
    y = half * (y + x / y)
    y = half * (y + x / y)
    y = half * (y + x / y)
    return y


def _gather_body(u_idx, i_idx, u_tabT, i_tabT, stg_u, stg_i,
                 all_idx, wl_k, wl_cb, spmem, ring, batch, kidx,
                 sem, sem2):
    cid = lax.axis_index("c")
    sid = lax.axis_index("s")
    sc_lo = cid * SC_HALF
    n_c = jnp.minimum(NBLK - sc_lo, SC_HALF)
    base_n = n_c // NS
    ex = n_c - base_n * NS
    lo = sc_lo + sid * base_n + jnp.minimum(sid, ex)
    nblk = base_n + jnp.where(sid < ex, 1, 0)
    hi = lo + nblk
    lanes = lax.iota(jnp.int32, L)

    for tab_ref, stg_ref, idx_ref in (
        (u_tabT, stg_u, u_idx),
        (i_tabT, stg_i, i_idx),
    ):
        pltpu.sync_copy(idx_ref, all_idx)

        def scan(q, cur):
            v = all_idx[pl.ds(q * L, L)]
            blk = v >> 7
            m = (blk >= lo) & (blk < hi)
            n = plsc.all_reduce_population_count(m)[0]
            kv = q * L + lanes
            cb = (v & 127) | (blk << 8)
            plsc.store_compressed(wl_k.at[pl.ds(cur, L)], kv, mask=m)
            plsc.store_compressed(wl_cb.at[pl.ds(cur, L)], cb, mask=m)
            return cur + n

        nwl = lax.fori_loop(0, B // L, scan, jnp.int32(0))
        nwlv = (nwl + L - 1) // L

        def fire(wi):
            wst = jnp.minimum(lo + wi * RING, NBLK - RING)
            start = pl.multiple_of(wst * 128, 128)
            par = wi & 1
            for a in range(D // 8):
                pltpu.async_copy(
                    tab_ref.at[pl.ds(a * 8, 8), pl.ds(start, RING * 128)],
                    spmem.at[par, sid, a], sem)

        def drain(wi):
            par = wi & 1
            for a in range(D // 8):
                pltpu.make_async_copy(
                    tab_ref.at[pl.ds(0, 8), pl.ds(0, RING * 128)],
                    spmem.at[par, sid, a], sem).wait()

        @pl.when(lo < hi)
        def _():
            fire(jnp.int32(0))

        def window(wi, _):
            wb = lo + wi * RING
            wst = jnp.minimum(wb, NBLK - RING)
            we = jnp.minimum(wb + RING, hi)
            par = wi & 1

            @pl.when(wb < hi)
            def _():
                drain(wi)

                @pl.when(wb + RING < hi)
                def _():
                    fire(wi + 1)

                for a in range(D // 8):
                    pltpu.sync_copy(spmem.at[par, sid, a], ring.at[a])

                def scan_hits(q, _):
                    kv = wl_k[pl.ds(q * L, L)]
                    cb = wl_cb[pl.ds(q * L, L)]
                    blk = cb >> 8
                    valid = (q * L + lanes) < nwl
                    m = (blk >= wb) & (blk < we) & valid
                    n = plsc.all_reduce_population_count(m)[0]

                    @pl.when(n > 0)
                    def _():
                        sc_vec = jnp.where(
                            m, (blk - wst) * 128 + (cb & 255), 0)
                        ksel = jnp.where(m, kv, B + lanes)
                        for f in range(D):
                            av = jnp.full((L,), f // 8, jnp.int32)
                            bv = jnp.full((L,), f % 8, jnp.int32)
                            fv = jnp.full((L,), f, jnp.int32)
                            vals = plsc.load_gather(
                                ring, [av, bv, sc_vec])
                            plsc.store_scatter(batch, [lanes, fv], vals)
                        kidx[...] = ksel
                        pltpu.async_copy(
                            batch, stg_ref.at[kidx], sem2).wait()
                    return 0

                lax.fori_loop(0, nwlv, scan_hits, 0)
            return 0

        lax.fori_loop(0, MAXWIN, window, 0)


def _dist_body(stg_u, stg_i, out_hbm, buf_u, buf_i, out_v, sem_u, sem_i):
    wid = lax.axis_index("s") * NC + lax.axis_index("c")
    base = wid * ROWS_PER_W
    lanes = lax.iota(jnp.int32, L)
    CH = 256
    for ch in range(ROWS_PER_W // CH):
        cu = pltpu.async_copy(
            stg_u.at[pl.ds(base + ch * CH, CH)], buf_u, sem_u)
        ci = pltpu.async_copy(
            stg_i.at[pl.ds(base + ch * CH, CH)], buf_i, sem_i)
        cu.wait()
        ci.wait()

        def group(g, _, ch=ch):
            rv = g * L + lanes
            acc = jnp.zeros((L,), jnp.float32)
            for f in range(D):
                fv = jnp.full((L,), f, jnp.int32)
                u = plsc.load_gather(buf_u, [rv, fv])
                v = plsc.load_gather(buf_i, [rv, fv])
                d = (u - v) + jnp.float32(EPS)
                acc = acc + d * d
            out_v[pl.ds(ch * CH + g * L, L)] = _sqrt16(acc)
            return 0

        lax.fori_loop(0, CH // L, group, 0)

    pltpu.sync_copy(out_v, out_hbm.at[pl.ds(base, ROWS_PER_W)])


_MESH = plsc.VectorSubcoreMesh(core_axis_name="c", subcore_axis_name="s")
_PARAMS = pltpu.CompilerParams(needs_layout_passes=False)

_gather_phase = functools.partial(
    pl.kernel,
    mesh=_MESH,
    out_type=(
        jax.ShapeDtypeStruct((SROWS, 128), jnp.float32),
        jax.ShapeDtypeStruct((SROWS, 128), jnp.float32),
    ),
    compiler_params=_PARAMS,
    scratch_types=[
        pltpu.VMEM((B,), jnp.int32),
        pltpu.VMEM((B + L,), jnp.int32),
        pltpu.VMEM((B + L,), jnp.int32),
        pltpu.VMEM_SHARED((2, NS, D // 8, 8, RING * 128), jnp.float32),
        pltpu.VMEM((D // 8, 8, RING * 128), jnp.float32),
        pltpu.VMEM((L, 128), jnp.float32),
        pltpu.VMEM((L,), jnp.int32),
        pltpu.SemaphoreType.DMA,
        pltpu.SemaphoreType.DMA,
    ],
)(lambda u_idx, i_idx, u_tabT, i_tabT, stg_u, stg_i, *scratch:
  _gather_body(u_idx, i_idx, u_tabT, i_tabT, stg_u, stg_i, *scratch))

_dist_phase = functools.partial(
    pl.kernel,
    mesh=_MESH,
    out_type=jax.ShapeDtypeStruct((B,), jnp.float32),
    compiler_params=_PARAMS,
    scratch_types=[
        pltpu.VMEM((256, 128), jnp.float32),
        pltpu.VMEM((256, 128), jnp.float32),
        pltpu.VMEM((ROWS_PER_W,), jnp.float32),
        pltpu.SemaphoreType.DMA,
        pltpu.SemaphoreType.DMA,
    ],
)(lambda stg_u, stg_i, out, *scratch: _dist_body(stg_u, stg_i, out, *scratch))


def kernel(U, I, UEmb_weight, IEmb_weight):
    stg_u, stg_i = _gather_phase(U, I, UEmb_weight.T, IEmb_weight.T)
    return _dist_phase(stg_u, stg_i)


# speedup vs baseline: 1.1221x; 1.0006x over previous
import functools

import jax
import jax.numpy as jnp
from jax import lax
from jax.experimental import pallas as pl
from jax.experimental.pallas import tpu as pltpu
from jax.experimental.pallas import tpu_sc as plsc

D = 64
B = 16384
V = 1000000
L = 16
NC = 2
NS = 16
NW = NC * NS
ROWS_PER_W = B // NW
NBLK = (V + 127) // 128
SC_HALF = (NBLK + 1) // 2
RING = 3
_MAXBLK = (SC_HALF + NS - 1) // NS
MAXWIN = (_MAXBLK + RING - 1) // RING + 1
SROWS = B + 2 * L
EPS = 1e-6


def _sqrt16(x):
    i = plsc.bitcast(x, jnp.int32)
    y = plsc.bitcast((i >> 1) + jnp.int32(0x1FBD1DF5), jnp.float32)
    half = jnp.float32(0.5)
    y = half * (y + x / y)
    y = half * (y + x / y)
    y = half * (y + x / y)
    return y


def _gather_body(u_idx, i_idx, u_tabT, i_tabT, stg_u, stg_i,
                 all_idx, wl_k, wl_cb, spmem, ring, batch, kidx,
                 sem_a, sem_b, sem_c, sem_d, sem2):
    sems = (sem_a, sem_b, sem_c, sem_d)
    cid = lax.axis_index("c")
    sid = lax.axis_index("s")
    sc_lo = cid * SC_HALF
    n_c = jnp.minimum(NBLK - sc_lo, SC_HALF)
    base_n = n_c // NS
    ex = n_c - base_n * NS
    lo = sc_lo + sid * base_n + jnp.minimum(sid, ex)
    nblk = base_n + jnp.where(sid < ex, 1, 0)
    hi = lo + nblk
    lanes = lax.iota(jnp.int32, L)

    for tab_ref, stg_ref, idx_ref in (
        (u_tabT, stg_u, u_idx),
        (i_tabT, stg_i, i_idx),
    ):
        pltpu.sync_copy(idx_ref, all_idx)

        def scan(q, cur):
            v = all_idx[pl.ds(q * L, L)]
            blk = v >> 7
            m = (blk >= lo) & (blk < hi)
            n = plsc.all_reduce_population_count(m)[0]
            kv = q * L + lanes
            cb = (v & 127) | (blk << 8)
            plsc.store_compressed(wl_k.at[pl.ds(cur, L)], kv, mask=m)
            plsc.store_compressed(wl_cb.at[pl.ds(cur, L)], cb, mask=m)
            return cur + n

        nwl = lax.fori_loop(0, B // L, scan, jnp.int32(0))
        nwlv = (nwl + L - 1) // L

        def fire(wi):
            wst = jnp.minimum(lo + wi * RING, NBLK - RING)
            start = pl.multiple_of(wst * 128, 128)
            par = wi & 1
            for a in range(D // 8):
                pltpu.async_copy(
                    tab_ref.at[pl.ds(a * 8, 8), pl.ds(start, RING * 128)],
                    spmem.at[par, sid, a], sems[a % 4])

        def drain(wi):
            par = wi & 1
            for a in range(D // 8):
                pltpu.make_async_copy(
                    tab_ref.at[pl.ds(0, 8), pl.ds(0, RING * 128)],
                    spmem.at[par, sid, a], sems[a % 4]).wait()

        @pl.when(lo < hi)
        def _():
            fire(jnp.int32(0))

        def window(wi, _):
            wb = lo + wi * RING
            wst = jnp.minimum(wb, NBLK - RING)
            we = jnp.minimum(wb + RING, hi)
            par = wi & 1

            @pl.when(wb < hi)
            def _():
                drain(wi)

                @pl.when(wb + RING < hi)
                def _():
                    fire(wi + 1)

                for a in range(D // 8):
                    pltpu.sync_copy(spmem.at[par, sid, a], ring.at[a])

                def scan_hits(q, _):
                    kv = wl_k[pl.ds(q * L, L)]
                    cb = wl_cb[pl.ds(q * L, L)]
                    blk = cb >> 8
                    valid = (q * L + lanes) < nwl
                    m = (blk >= wb) & (blk < we) & valid
                    n = plsc.all_reduce_population_count(m)[0]

                    @pl.when(n > 0)
                    def _():
                        sc_vec = jnp.where(
                            m, (blk - wst) * 128 + (cb & 255), 0)
                        ksel = jnp.where(m, kv, B + lanes)
                        for f in range(D):
                            av = jnp.full((L,), f // 8, jnp.int32)
                            bv = jnp.full((L,), f % 8, jnp.int32)
                            fv = jnp.full((L,), f, jnp.int32)
                            vals = plsc.load_gather(
                                ring, [av, bv, sc_vec])
                            plsc.store_scatter(batch, [lanes, fv], vals)
                        kidx[...] = ksel
                        pltpu.async_copy(
                            batch, stg_ref.at[kidx], sem2).wait()
                    return 0

                lax.fori_loop(0, nwlv, scan_hits, 0)
            return 0

        lax.fori_loop(0, MAXWIN, window, 0)


def _dist_body(stg_u, stg_i, out_hbm, buf_u, buf_i, out_v, sem_u, sem_i):
    wid = lax.axis_index("s") * NC + lax.axis_index("c")
    base = wid * ROWS_PER_W
    lanes = lax.iota(jnp.int32, L)
    CH = 256
    for ch in range(ROWS_PER_W // CH):
        cu = pltpu.async_copy(
            stg_u.at[pl.ds(base + ch * CH, CH)], buf_u, sem_u)
        ci = pltpu.async_copy(
            stg_i.at[pl.ds(base + ch * CH, CH)], buf_i, sem_i)
        cu.wait()
        ci.wait()

        def group(g, _, ch=ch):
            rv = g * L + lanes
            acc = jnp.zeros((L,), jnp.float32)
            for f in range(D):
                fv = jnp.full((L,), f, jnp.int32)
                u = plsc.load_gather(buf_u, [rv, fv])
                v = plsc.load_gather(buf_i, [rv, fv])
                d = (u - v) + jnp.float32(EPS)
                acc = acc + d * d
            out_v[pl.ds(ch * CH + g * L, L)] = _sqrt16(acc)
            return 0

        lax.fori_loop(0, CH // L, group, 0)

    pltpu.sync_copy(out_v, out_hbm.at[pl.ds(base, ROWS_PER_W)])


_MESH = plsc.VectorSubcoreMesh(core_axis_name="c", subcore_axis_name="s")
_PARAMS = pltpu.CompilerParams(needs_layout_passes=False)

_gather_phase = functools.partial(
    pl.kernel,
    mesh=_MESH,
    out_type=(
        jax.ShapeDtypeStruct((SROWS, 128), jnp.float32),
        jax.ShapeDtypeStruct((SROWS, 128), jnp.float32),
    ),
    compiler_params=_PARAMS,
    scratch_types=[
        pltpu.VMEM((B,), jnp.int32),
        pltpu.VMEM((B + L,), jnp.int32),
        pltpu.VMEM((B + L,), jnp.int32),
        pltpu.VMEM_SHARED((2, NS, D // 8, 8, RING * 128), jnp.float32),
        pltpu.VMEM((D // 8, 8, RING * 128), jnp.float32),
        pltpu.VMEM((L, 128), jnp.float32),
        pltpu.VMEM((L,), jnp.int32),
        pltpu.SemaphoreType.DMA,
        pltpu.SemaphoreType.DMA,
        pltpu.SemaphoreType.DMA,
        pltpu.SemaphoreType.DMA,
        pltpu.SemaphoreType.DMA,
    ],
)(lambda u_idx, i_idx, u_tabT, i_tabT, stg_u, stg_i, *scratch:
  _gather_body(u_idx, i_idx, u_tabT, i_tabT, stg_u, stg_i, *scratch))

_dist_phase = functools.partial(
    pl.kernel,
    mesh=_MESH,
    out_type=jax.ShapeDtypeStruct((B,), jnp.float32),
    compiler_params=_PARAMS,
    scratch_types=[
        pltpu.VMEM((256, 128), jnp.float32),
        pltpu.VMEM((256, 128), jnp.float32),
        pltpu.VMEM((ROWS_PER_W,), jnp.float32),
        pltpu.SemaphoreType.DMA,
        pltpu.SemaphoreType.DMA,
    ],
)(lambda stg_u, stg_i, out, *scratch: _dist_body(stg_u, stg_i, out, *scratch))


def kernel(U, I, UEmb_weight, IEmb_weight):
    stg_u, stg_i = _gather_phase(U, I, UEmb_weight.T, IEmb_weight.T)
    return _dist_phase(stg_u, stg_i)


# speedup vs baseline: 1.5782x; 1.4064x over previous
import functools

import jax
import jax.numpy as jnp
from jax import lax
from jax.experimental import pallas as pl
from jax.experimental.pallas import tpu as pltpu
from jax.experimental.pallas import tpu_sc as plsc

D = 64
B = 16384
L = 16
NC = 2
NS = 16
NW = NC * NS
ROWS_PER_W = B // NW
CHUNK = 128
NCHUNK = ROWS_PER_W // CHUNK
GROUPS = CHUNK // L
EPS = 1e-6


def _sqrt16(x):
    i = plsc.bitcast(x, jnp.int32)
    y = plsc.bitcast((i >> 1) + jnp.int32(0x1FBD1DF5), jnp.float32)
    half = jnp.float32(0.5)
    y = half * (y + x / y)
    y = half * (y + x / y)
    y = half * (y + x / y)
    return y


def _body(u_idx_hbm, i_idx_hbm, u_tab, i_tab, out_hbm,
          idx_u_v, idx_i_v, pair_u_v, pair_i_v, rows_u, rows_i, out_v,
          sem_u, sem_i):
    wid = lax.axis_index("s") * NC + lax.axis_index("c")
    blk = wid * NCHUNK
    pltpu.sync_copy(u_idx_hbm.at[pl.ds(blk, NCHUNK)], idx_u_v)
    pltpu.sync_copy(i_idx_hbm.at[pl.ds(blk, NCHUNK)], idx_i_v)

    def to_pairs(k, _):
        for c in range(NCHUNK):
            sl = pl.ds(k * L, L)
            pair_u_v[c, sl] = idx_u_v[c, sl] >> 1
            pair_i_v[c, sl] = idx_i_v[c, sl] >> 1
        return 0

    lax.fori_loop(0, GROUPS, to_pairs, 0)
    lanes = lax.iota(jnp.int32, L)

    for c in range(NCHUNK):
        cu = pltpu.async_copy(u_tab.at[pair_u_v.at[c]], rows_u, sem_u)
        ci = pltpu.async_copy(i_tab.at[pair_i_v.at[c]], rows_i, sem_i)
        cu.wait()
        ci.wait()

        def group(g, _, c=c):
            ridx = g * L + lanes
            sl = pl.ds(g * L, L)
            iu = idx_u_v[c, sl]
            ii = idx_i_v[c, sl]
            off_u = (iu & 1) << 6
            off_i = (ii & 1) << 6
            acc = jnp.zeros((L,), jnp.float32)
            for j in range(D):
                jv = jnp.full((L,), j, jnp.int32)
                u = plsc.load_gather(rows_u, [ridx, off_u + jv])
                v = plsc.load_gather(rows_i, [ridx, off_i + jv])
                d = (u - v) + jnp.float32(EPS)
                acc = acc + d * d
            out_v[pl.ds(c * CHUNK + g * L, L)] = _sqrt16(acc)
            return 0

        lax.fori_loop(0, GROUPS, group, 0)

    base = wid * ROWS_PER_W
    pltpu.sync_copy(out_v, out_hbm.at[pl.ds(base, ROWS_PER_W)])


@functools.partial(
    pl.kernel,
    mesh=plsc.VectorSubcoreMesh(core_axis_name="c", subcore_axis_name="s"),
    out_type=jax.ShapeDtypeStruct((B,), jnp.float32),
    compiler_params=pltpu.CompilerParams(needs_layout_passes=False),
    scratch_types=[
        pltpu.VMEM((NCHUNK, CHUNK), jnp.int32),
        pltpu.VMEM((NCHUNK, CHUNK), jnp.int32),
        pltpu.VMEM((NCHUNK, CHUNK), jnp.int32),
        pltpu.VMEM((NCHUNK, CHUNK), jnp.int32),
        pltpu.VMEM((CHUNK, 2 * D), jnp.float32),
        pltpu.VMEM((CHUNK, 2 * D), jnp.float32),
        pltpu.VMEM((ROWS_PER_W,), jnp.float32),
        pltpu.SemaphoreType.DMA,
        pltpu.SemaphoreType.DMA,
    ],
)
def _cml_dist(u_idx, i_idx, u_tab, i_tab, out, *scratch):
    _body(u_idx, i_idx, u_tab, i_tab, out, *scratch)


def kernel(U, I, UEmb_weight, IEmb_weight):
    U2 = U.reshape(NW * NCHUNK, CHUNK)
    I2 = I.reshape(NW * NCHUNK, CHUNK)
    UT = UEmb_weight.reshape(-1, 2 * D)
    IT = IEmb_weight.reshape(-1, 2 * D)
    return _cml_dist(U2, I2, UT, IT)
